# Initial kernel scaffold; baseline (speedup 1.0000x reference)
import jax
import jax.numpy as jnp
from jax.experimental import pallas as pl


def kernel(x, Wr, br, W1, W3, W2):
    raise NotImplementedError("write your pallas kernel here")


# speedup vs baseline: 1.1254x; 1.1254x over previous
import functools

import jax
import jax.numpy as jnp
from jax import lax
from jax.experimental import pallas as pl
from jax.experimental.pallas import tpu as pltpu
from jax.experimental.pallas import tpu_sc as plsc

S = 2048
H = 768
F = 2048
NE = 8
TOPK = 2
A = S * TOPK
T = 256
APAD = A + NE * T
GMAX = A // T + NE
EPAD = 128
NC, NS, L = 2, 16, 16
NW = NC * NS


def _router_body(x_ref, wrt_ref, br_ref, i1_ref, i2_ref, w1_ref, w2_ref,
                 cnt_ref):
    logits = jnp.dot(x_ref[...], wrt_ref[...],
                     preferred_element_type=jnp.float32) + br_ref[...]
    lanes = lax.broadcasted_iota(jnp.int32, (S, EPAD), 1)
    m1 = jnp.max(logits, axis=1, keepdims=True)
    a1 = jnp.min(jnp.where(logits == m1, lanes, EPAD), axis=1, keepdims=True)
    l2 = jnp.where(lanes == a1, -jnp.inf, logits)
    m2 = jnp.max(l2, axis=1, keepdims=True)
    a2 = jnp.min(jnp.where(l2 == m2, lanes, EPAD), axis=1, keepdims=True)
    ed = jnp.exp(m2 - m1)
    w1 = 1.0 / (1.0 + ed)
    i1_ref[...] = a1
    i2_ref[...] = a2
    w1_ref[...] = w1
    w2_ref[...] = ed * w1
    onehot = (lanes == a1).astype(jnp.int32) + (lanes == a2).astype(jnp.int32)
    cnt_ref[...] = jnp.sum(onehot, axis=0, keepdims=True)


def _router(x2d, Wr, br):
    wrt = jnp.zeros((H, EPAD), jnp.float32).at[:, :NE].set(Wr.T)
    brp = jnp.full((1, EPAD), -1e30, jnp.float32).at[0, :NE].set(br)
    return pl.pallas_call(
        _router_body,
        out_shape=[
            jax.ShapeDtypeStruct((S, 1), jnp.int32),
            jax.ShapeDtypeStruct((S, 1), jnp.int32),
            jax.ShapeDtypeStruct((S, 1), jnp.float32),
            jax.ShapeDtypeStruct((S, 1), jnp.float32),
            jax.ShapeDtypeStruct((1, EPAD), jnp.int32),
        ],
    )(x2d, wrt, brp)


def _make_sc_gather():
    bpw = APAD // NW
    chunk = bpw // 2
    mesh = plsc.VectorSubcoreMesh(core_axis_name="c", subcore_axis_name="s")

    @functools.partial(
        pl.kernel, mesh=mesh,
        out_type=jax.ShapeDtypeStruct((APAD, H), jnp.float32),
        scratch_types=[
            pltpu.VMEM((chunk,), jnp.int32),
            pltpu.VMEM((chunk, H), jnp.float32),
            pltpu.SemaphoreType.DMA,
        ],
    )
    def gather_k(x_hbm, st_hbm, out_hbm, idx_v, rows_v, sem):
        wid = lax.axis_index("s") * NC + lax.axis_index("c")
        for c in range(2):
            base = wid * bpw + c * chunk
            pltpu.sync_copy(st_hbm.at[pl.ds(base, chunk)], idx_v)
            pltpu.async_copy(x_hbm.at[idx_v], rows_v, sem).wait()
            pltpu.sync_copy(rows_v, out_hbm.at[pl.ds(base, chunk)])

    return gather_k


def _ffn_body(eid_ref, rblk_ref, gc_ref, xs_ref, sw_ref, w1_ref, w3_ref,
              w2_ref, ys_ref):
    g = pl.program_id(0)

    @pl.when(g < gc_ref[0])
    def _():
        xb = xs_ref[...]
        h1 = jnp.dot(xb, w1_ref[0], preferred_element_type=jnp.float32)
        h3 = jnp.dot(xb, w3_ref[0], preferred_element_type=jnp.float32)
        hid = h1 * jax.nn.sigmoid(h1) * h3
        yb = jnp.dot(hid, w2_ref[0], preferred_element_type=jnp.float32)
        ys_ref[...] = yb * sw_ref[...]


def _ffn(eid, rblk, gcount, xs, swp, W1, W3, W2):
    grid_spec = pltpu.PrefetchScalarGridSpec(
        num_scalar_prefetch=3,
        grid=(GMAX,),
        in_specs=[
            pl.BlockSpec((T, H), lambda g, eid, rblk, gc: (rblk[g], 0)),
            pl.BlockSpec((T, 1), lambda g, eid, rblk, gc: (rblk[g], 0)),
            pl.BlockSpec((1, H, F), lambda g, eid, rblk, gc: (eid[g], 0, 0)),
            pl.BlockSpec((1, H, F), lambda g, eid, rblk, gc: (eid[g], 0, 0)),
            pl.BlockSpec((1, F, H), lambda g, eid, rblk, gc: (eid[g], 0, 0)),
        ],
        out_specs=pl.BlockSpec((T, H), lambda g, eid, rblk, gc: (rblk[g], 0)),
    )
    return pl.pallas_call(
        _ffn_body,
        grid_spec=grid_spec,
        out_shape=jax.ShapeDtypeStruct((APAD, H), jnp.float32),
        compiler_params=pltpu.CompilerParams(
            dimension_semantics=("arbitrary",)),
    )(eid, rblk, gcount, xs, swp, W1, W3, W2)


def _make_sc_combine():
    tpw = S // NW
    mesh = plsc.VectorSubcoreMesh(core_axis_name="c", subcore_axis_name="s")

    @functools.partial(
        pl.kernel, mesh=mesh,
        out_type=jax.ShapeDtypeStruct((S, H), jnp.float32),
        scratch_types=[
            pltpu.VMEM((tpw,), jnp.int32),
            pltpu.VMEM((tpw,), jnp.int32),
            pltpu.VMEM((tpw, H), jnp.float32),
            pltpu.VMEM((tpw, H), jnp.float32),
            pltpu.SemaphoreType.DMA,
            pltpu.SemaphoreType.DMA,
        ],
    )
    def combine_k(ys_hbm, p1_hbm, p2_hbm, out_hbm, i1_v, i2_v, r1_v, r2_v,
                  sem1, sem2):
        wid = lax.axis_index("s") * NC + lax.axis_index("c")
        base = wid * tpw
        pltpu.sync_copy(p1_hbm.at[pl.ds(base, tpw)], i1_v)
        pltpu.sync_copy(p2_hbm.at[pl.ds(base, tpw)], i2_v)
        cp1 = pltpu.async_copy(ys_hbm.at[i1_v], r1_v, sem1)
        cp2 = pltpu.async_copy(ys_hbm.at[i2_v], r2_v, sem2)
        cp1.wait()
        cp2.wait()

        def _row(r, carry):
            for c in range(H // L):
                sl = pl.ds(c * L, L)
                r1_v[r, sl] = r1_v[r, sl] + r2_v[r, sl]
            return carry

        lax.fori_loop(0, tpw, _row, 0)
        pltpu.sync_copy(r1_v, out_hbm.at[pl.ds(base, tpw)])

    return combine_k


_sc_gather = _make_sc_gather()
_sc_combine = _make_sc_combine()


def _routing_metadata(i1, i2, w1, w2, cnt):
    e_all = jnp.concatenate([i1, i2])
    order = jnp.argsort(e_all, stable=True).astype(jnp.int32)
    counts = cnt[0, :NE]
    off = jnp.concatenate([jnp.zeros((1,), jnp.int32),
                           jnp.cumsum(counts).astype(jnp.int32)])
    nblk = (counts + (T - 1)) // T
    cum_nblk = jnp.cumsum(nblk).astype(jnp.int32)
    gstart = cum_nblk - nblk
    gcount = cum_nblk[-1:]
    off_pad = jnp.concatenate([jnp.zeros((1,), jnp.int32),
                               jnp.cumsum(nblk * T).astype(jnp.int32)])
    gs = jnp.arange(GMAX, dtype=jnp.int32)
    g_eff = jnp.minimum(gs, gcount[0] - 1)
    eid = jnp.searchsorted(cum_nblk, g_eff, side="right").astype(jnp.int32)
    rblk = off_pad[eid] // T + (g_eff - gstart[eid])
    inv = jnp.zeros((A,), jnp.int32).at[order].set(
        jnp.arange(A, dtype=jnp.int32))
    pos = off_pad[e_all] + (inv - off[e_all])
    t_all = jnp.concatenate([jnp.arange(S, dtype=jnp.int32)] * 2)
    st_pad = jnp.zeros((APAD,), jnp.int32).at[pos].set(t_all)
    w_all = jnp.concatenate([w1, w2])
    sw_pad = jnp.zeros((APAD, 1), jnp.float32).at[pos, 0].set(w_all)
    p1, p2 = pos[:S], pos[S:]
    return st_pad, sw_pad, eid, rblk, gcount, p1, p2


def kernel(x, Wr, br, W1, W3, W2):
    x2d = x.reshape(S, H)
    i1c, i2c, w1c, w2c, cnt = _router(x2d, Wr, br)
    st_pad, sw_pad, eid, rblk, gcount, p1, p2 = _routing_metadata(
        i1c[:, 0], i2c[:, 0], w1c[:, 0], w2c[:, 0], cnt)
    xs = _sc_gather(x2d, st_pad)
    ys = _ffn(eid, rblk, gcount, xs, sw_pad, W1, W3, W2)
    out = _sc_combine(ys, p1, p2)
    return out.reshape(x.shape)


# speedup vs baseline: 1.2516x; 1.1121x over previous
import functools

import jax
import jax.numpy as jnp
from jax import lax
from jax.experimental import pallas as pl
from jax.experimental.pallas import tpu as pltpu
from jax.experimental.pallas import tpu_sc as plsc

S = 2048
H = 768
F = 2048
NE = 8
TOPK = 2
A = S * TOPK
T = 128
APAD = A + NE * T
GMAX = A // T + NE
EPAD = 128
NC, NS, L = 2, 16, 16
NW = NC * NS


def _router_body(x_ref, wrt_ref, br_ref, i1_ref, i2_ref, w1_ref, w2_ref,
                 cnt_ref):
    logits = jnp.dot(x_ref[...], wrt_ref[...],
                     preferred_element_type=jnp.float32) + br_ref[...]
    lanes = lax.broadcasted_iota(jnp.int32, (S, EPAD), 1)
    m1 = jnp.max(logits, axis=1, keepdims=True)
    a1 = jnp.min(jnp.where(logits == m1, lanes, EPAD), axis=1, keepdims=True)
    l2 = jnp.where(lanes == a1, -jnp.inf, logits)
    m2 = jnp.max(l2, axis=1, keepdims=True)
    a2 = jnp.min(jnp.where(l2 == m2, lanes, EPAD), axis=1, keepdims=True)
    ed = jnp.exp(m2 - m1)
    w1 = 1.0 / (1.0 + ed)
    i1_ref[...] = a1
    i2_ref[...] = a2
    w1_ref[...] = w1
    w2_ref[...] = ed * w1
    onehot = (lanes == a1).astype(jnp.int32) + (lanes == a2).astype(jnp.int32)
    cnt_ref[...] = jnp.sum(onehot, axis=0, keepdims=True)


def _router(x2d, Wr, br):
    wrt = jnp.zeros((H, EPAD), jnp.float32).at[:, :NE].set(Wr.T)
    brp = jnp.full((1, EPAD), -1e30, jnp.float32).at[0, :NE].set(br)
    return pl.pallas_call(
        _router_body,
        out_shape=[
            jax.ShapeDtypeStruct((S, 1), jnp.int32),
            jax.ShapeDtypeStruct((S, 1), jnp.int32),
            jax.ShapeDtypeStruct((S, 1), jnp.float32),
            jax.ShapeDtypeStruct((S, 1), jnp.float32),
            jax.ShapeDtypeStruct((1, EPAD), jnp.int32),
        ],
    )(x2d, wrt, brp)


def _make_sc_gather():
    bpw = APAD // NW
    chunk = bpw // 2
    mesh = plsc.VectorSubcoreMesh(core_axis_name="c", subcore_axis_name="s")

    @functools.partial(
        pl.kernel, mesh=mesh,
        out_type=jax.ShapeDtypeStruct((APAD, H), jnp.float32),
        scratch_types=[
            pltpu.VMEM((chunk,), jnp.int32),
            pltpu.VMEM((chunk,), jnp.int32),
            pltpu.VMEM((chunk, H), jnp.float32),
            pltpu.VMEM((chunk, H), jnp.float32),
            pltpu.SemaphoreType.DMA,
            pltpu.SemaphoreType.DMA,
            pltpu.SemaphoreType.DMA,
            pltpu.SemaphoreType.DMA,
        ],
    )
    def gather_k(x_hbm, st_hbm, out_hbm, idx0, idx1, rows0, rows1,
                 sg0, sg1, sw0, sw1):
        wid = lax.axis_index("s") * NC + lax.axis_index("c")
        b0 = wid * bpw
        b1 = b0 + chunk
        pltpu.sync_copy(st_hbm.at[pl.ds(b0, chunk)], idx0)
        pltpu.sync_copy(st_hbm.at[pl.ds(b1, chunk)], idx1)
        g0 = pltpu.async_copy(x_hbm.at[idx0], rows0, sg0)
        g1 = pltpu.async_copy(x_hbm.at[idx1], rows1, sg1)
        g0.wait()
        w0 = pltpu.async_copy(rows0, out_hbm.at[pl.ds(b0, chunk)], sw0)
        g1.wait()
        w1 = pltpu.async_copy(rows1, out_hbm.at[pl.ds(b1, chunk)], sw1)
        w0.wait()
        w1.wait()

    return gather_k


def _ffn_body(eid_ref, rblk_ref, gc_ref, xs_ref, sw_ref, w1_ref, w3_ref,
              w2_ref, ys_ref):
    g = pl.program_id(0)

    @pl.when(g < gc_ref[0])
    def _():
        xb = xs_ref[...]
        h1 = jnp.dot(xb, w1_ref[0], preferred_element_type=jnp.float32)
        h3 = jnp.dot(xb, w3_ref[0], preferred_element_type=jnp.float32)
        hid = h1 * jax.nn.sigmoid(h1) * h3
        yb = jnp.dot(hid, w2_ref[0], preferred_element_type=jnp.float32)
        ys_ref[...] = yb * sw_ref[...]


def _ffn(eid, rblk, gcount, xs, swp, W1, W3, W2):
    grid_spec = pltpu.PrefetchScalarGridSpec(
        num_scalar_prefetch=3,
        grid=(GMAX,),
        in_specs=[
            pl.BlockSpec((T, H), lambda g, eid, rblk, gc: (rblk[g], 0)),
            pl.BlockSpec((T, 1), lambda g, eid, rblk, gc: (rblk[g], 0)),
            pl.BlockSpec((1, H, F), lambda g, eid, rblk, gc: (eid[g], 0, 0)),
            pl.BlockSpec((1, H, F), lambda g, eid, rblk, gc: (eid[g], 0, 0)),
            pl.BlockSpec((1, F, H), lambda g, eid, rblk, gc: (eid[g], 0, 0)),
        ],
        out_specs=pl.BlockSpec((T, H), lambda g, eid, rblk, gc: (rblk[g], 0)),
    )
    return pl.pallas_call(
        _ffn_body,
        grid_spec=grid_spec,
        out_shape=jax.ShapeDtypeStruct((APAD, H), jnp.float32),
        compiler_params=pltpu.CompilerParams(
            dimension_semantics=("arbitrary",)),
    )(eid, rblk, gcount, xs, swp, W1, W3, W2)


def _make_sc_combine():
    tpw = S // NW
    mesh = plsc.VectorSubcoreMesh(core_axis_name="c", subcore_axis_name="s")

    @functools.partial(
        pl.kernel, mesh=mesh,
        out_type=jax.ShapeDtypeStruct((S, H), jnp.float32),
        scratch_types=[
            pltpu.VMEM((tpw,), jnp.int32),
            pltpu.VMEM((tpw,), jnp.int32),
            pltpu.VMEM((tpw, H), jnp.float32),
            pltpu.VMEM((tpw, H), jnp.float32),
            pltpu.SemaphoreType.DMA,
            pltpu.SemaphoreType.DMA,
        ],
    )
    def combine_k(ys_hbm, p1_hbm, p2_hbm, out_hbm, i1_v, i2_v, r1_v, r2_v,
                  sem1, sem2):
        wid = lax.axis_index("s") * NC + lax.axis_index("c")
        base = wid * tpw
        pltpu.sync_copy(p1_hbm.at[pl.ds(base, tpw)], i1_v)
        pltpu.sync_copy(p2_hbm.at[pl.ds(base, tpw)], i2_v)
        cp1 = pltpu.async_copy(ys_hbm.at[i1_v], r1_v, sem1)
        cp2 = pltpu.async_copy(ys_hbm.at[i2_v], r2_v, sem2)
        cp1.wait()
        cp2.wait()

        def _row(r, carry):
            for c in range(H // L):
                sl = pl.ds(c * L, L)
                r1_v[r, sl] = r1_v[r, sl] + r2_v[r, sl]
            return carry

        lax.fori_loop(0, tpw, _row, 0)
        pltpu.sync_copy(r1_v, out_hbm.at[pl.ds(base, tpw)])

    return combine_k


_sc_gather = _make_sc_gather()
_sc_combine = _make_sc_combine()


def _routing_metadata(i1, i2, w1, w2, cnt):
    e_all = jnp.concatenate([i1, i2])
    order = jnp.argsort(e_all, stable=True).astype(jnp.int32)
    counts = cnt[0, :NE]
    off = jnp.concatenate([jnp.zeros((1,), jnp.int32),
                           jnp.cumsum(counts).astype(jnp.int32)])
    nblk = (counts + (T - 1)) // T
    cum_nblk = jnp.cumsum(nblk).astype(jnp.int32)
    gstart = cum_nblk - nblk
    gcount = cum_nblk[-1:]
    off_pad = jnp.concatenate([jnp.zeros((1,), jnp.int32),
                               jnp.cumsum(nblk * T).astype(jnp.int32)])
    gs = jnp.arange(GMAX, dtype=jnp.int32)
    g_eff = jnp.minimum(gs, gcount[0] - 1)
    eid = jnp.searchsorted(cum_nblk, g_eff, side="right").astype(jnp.int32)
    rblk = off_pad[eid] // T + (g_eff - gstart[eid])
    inv = jnp.zeros((A,), jnp.int32).at[order].set(
        jnp.arange(A, dtype=jnp.int32))
    pos = off_pad[e_all] + (inv - off[e_all])
    t_all = jnp.concatenate([jnp.arange(S, dtype=jnp.int32)] * 2)
    st_pad = jnp.zeros((APAD,), jnp.int32).at[pos].set(t_all)
    w_all = jnp.concatenate([w1, w2])
    sw_pad = jnp.zeros((APAD, 1), jnp.float32).at[pos, 0].set(w_all)
    p1, p2 = pos[:S], pos[S:]
    return st_pad, sw_pad, eid, rblk, gcount, p1, p2


def kernel(x, Wr, br, W1, W3, W2):
    x2d = x.reshape(S, H)
    i1c, i2c, w1c, w2c, cnt = _router(x2d, Wr, br)
    st_pad, sw_pad, eid, rblk, gcount, p1, p2 = _routing_metadata(
        i1c[:, 0], i2c[:, 0], w1c[:, 0], w2c[:, 0], cnt)
    xs = _sc_gather(x2d, st_pad)
    ys = _ffn(eid, rblk, gcount, xs, sw_pad, W1, W3, W2)
    out = _sc_combine(ys, p1, p2)
    return out.reshape(x.shape)
